# Initial kernel scaffold; baseline (speedup 1.0000x reference)
import jax
import jax.numpy as jnp
from jax.experimental import pallas as pl


def kernel(_x_):
    raise NotImplementedError("write your pallas kernel here")


# speedup vs baseline: 1.3558x; 1.3558x over previous
import numpy as np
import jax
import jax.numpy as jnp
from jax.experimental import pallas as pl
from jax.experimental.pallas import tpu as pltpu

B, C, H, W = 4, 96, 224, 224
BC = B * C
HW = H * W
T = 5
KSIZE = 7
SIGMA = 1.5
PAD = KSIZE // 2


def _np_threefry2x32(k1, k2, x0, x1):
    x0 = np.uint32(x0)
    x1 = np.uint32(x1)
    ks0 = np.uint32(k1)
    ks1 = np.uint32(k2)
    ks2 = np.uint32(ks0 ^ ks1 ^ np.uint32(0x1BD11BDA))
    with np.errstate(over="ignore"):
        x0 = (x0 + ks0).astype(np.uint32)
        x1 = (x1 + ks1).astype(np.uint32)
        rots = [[13, 15, 26, 6], [17, 29, 16, 24]]
        kadd = [(ks1, ks2, 1), (ks2, ks0, 2), (ks0, ks1, 3),
                (ks1, ks2, 4), (ks2, ks0, 5)]
        for i in range(5):
            for r in rots[i % 2]:
                x0 = (x0 + x1).astype(np.uint32)
                x1 = ((x1 << np.uint32(r)) | (x1 >> np.uint32(32 - r))).astype(np.uint32)
                x1 = (x1 ^ x0).astype(np.uint32)
            a, b, c = kadd[i]
            x0 = (x0 + a).astype(np.uint32)
            x1 = (x1 + b + np.uint32(c)).astype(np.uint32)
    return x0, x1


_KEYS = [_np_threefry2x32(np.uint32(0), np.uint32(42), np.uint32(0), np.uint32(t))
         for t in range(T)]

_xs = np.linspace(-PAD, PAD, KSIZE, dtype=np.float32)
_pdf = np.exp(np.float32(-0.5) * (_xs / np.float32(SIGMA)) ** 2, dtype=np.float32)
_K1TAPS = (_pdf / _pdf.sum(dtype=np.float32)).astype(np.float32)


def _rotl(x, d):
    return (x << jnp.uint32(d)) | (x >> jnp.uint32(32 - d))


def _threefry_bits(k1, k2, lo):
    ks0 = jnp.uint32(k1)
    ks1 = jnp.uint32(k2)
    ks2 = jnp.uint32(int(np.uint32(k1) ^ np.uint32(k2) ^ np.uint32(0x1BD11BDA)))
    x0 = jnp.full_like(lo, ks0)
    x1 = lo + ks1
    rots = [[13, 15, 26, 6], [17, 29, 16, 24]]
    kadd = [(ks1, ks2, 1), (ks2, ks0, 2), (ks0, ks1, 3),
            (ks1, ks2, 4), (ks2, ks0, 5)]
    for i in range(5):
        for r in rots[i % 2]:
            x0 = x0 + x1
            x1 = _rotl(x1, r)
            x1 = x1 ^ x0
        a, b, c = kadd[i]
        x0 = x0 + a
        x1 = x1 + b + jnp.uint32(c)
    return x0 ^ x1


def _sample_kernel(x_ref, maxs_ref, idxs_ref):
    pid = pl.program_id(0)
    sig = jax.nn.sigmoid(x_ref[0])
    row = jax.lax.broadcasted_iota(jnp.int32, (H, W), 0)
    col = jax.lax.broadcasted_iota(jnp.int32, (H, W), 1)
    plane_idx = row * W + col
    lo = (pid * HW + plane_idx).astype(jnp.uint32)
    lane = jax.lax.broadcasted_iota(jnp.int32, (1, 128), 1)
    mvec = jnp.zeros((1, 128), jnp.float32)
    ivec = jnp.zeros((1, 128), jnp.int32)
    for t in range(T):
        k1, k2 = _KEYS[t]
        bits = _threefry_bits(int(k1), int(k2), lo)
        u = jax.lax.bitcast_convert_type(
            (bits >> jnp.uint32(9)) | jnp.uint32(0x3F800000), jnp.float32) - 1.0
        crnd = sig * u
        m_t = jnp.max(crnd)
        am_t = jnp.min(jnp.where(crnd == m_t, plane_idx, HW))
        mvec = jnp.where(lane == t, m_t, mvec)
        ivec = jnp.where(lane == t, am_t, ivec)
    maxs_ref[0] = mvec
    idxs_ref[0] = ivec


def _taps_at(a):
    out = jnp.zeros(a.shape, jnp.float32)
    for j in range(KSIZE):
        out = out + jnp.where(a == j, jnp.float32(_K1TAPS[j]), 0.0)
    return out


def _blur_vec(r, n):
    y = jax.lax.broadcasted_iota(jnp.int32, (BC, n), 1)
    v = _taps_at(r + PAD - y)
    top = (r >= 1) & (r <= PAD)
    v = v + jnp.where(top, _taps_at(PAD - r - y), 0.0)
    bot = (r >= n - 1 - PAD) & (r <= n - 2)
    v = v + jnp.where(bot, _taps_at(2 * n + 1 - r - y), 0.0)
    return v


def _stats_kernel(maxs_ref, idxs_ref, stats_ref, rv_ref, cv_ref):
    maxs = maxs_ref[...]
    idxs = idxs_ref[...]
    lane = jax.lax.broadcasted_iota(jnp.int32, (BC, 128), 1)
    tmask = lane < T
    vis_t = jnp.where(tmask & (maxs >= 0.1), 1.0, 0.0)
    i_t = (idxs // W).astype(jnp.float32) / H
    j_t = (idxs % W).astype(jnp.float32) / W
    yi = i_t * vis_t
    yj = j_t * vis_t
    m_i = jnp.sum(yi, axis=1, keepdims=True) / T
    m_j = jnp.sum(yj, axis=1, keepdims=True) / T
    di = jnp.where(tmask, yi - m_i, 0.0)
    dj = jnp.where(tmask, yj - m_j, 0.0)
    v_i = jnp.sum(di * di, axis=1, keepdims=True) / (T - 1)
    v_j = jnp.sum(dj * dj, axis=1, keepdims=True) / (T - 1)
    cnt = jnp.sum(vis_t, axis=1, keepdims=True)
    visf = jnp.where(cnt > 0.5 * T, 1.0, 0.0)
    m_i = m_i * visf
    m_j = m_j * visf
    v_i = v_i * visf
    v_j = v_j * visf
    rows = jnp.clip(jnp.round(m_i * H), 0, H - 1).astype(jnp.int32)
    cols = jnp.clip(jnp.round(m_j * W), 0, W - 1).astype(jnp.int32)
    rv = _blur_vec(rows, H)
    cv = _blur_vec(cols, W)
    chmax = visf * jnp.max(rv, axis=1, keepdims=True) * jnp.max(cv, axis=1, keepdims=True)
    gmax = jnp.max(chmax)
    inv = 1.0 / gmax
    rv_ref[...] = rv * visf * inv
    cv_ref[...] = cv
    stats = jnp.zeros((BC, 128), jnp.float32)
    for k, vec in enumerate((m_i, m_j, v_i, v_j, visf)):
        stats = jnp.where(lane == k, vec, stats)
    stats_ref[...] = stats


def _outer_kernel(rv_ref, cv_ref, att_ref):
    rcol = jnp.transpose(rv_ref[0])
    att_ref[0] = rcol * cv_ref[0]


@jax.jit
def _ips(x):
    xr = x.reshape(BC, H, W)
    maxs, idxs = pl.pallas_call(
        _sample_kernel,
        grid=(BC,),
        in_specs=[pl.BlockSpec((1, H, W), lambda i: (i, 0, 0))],
        out_specs=[
            pl.BlockSpec((1, 1, 128), lambda i: (i, 0, 0)),
            pl.BlockSpec((1, 1, 128), lambda i: (i, 0, 0)),
        ],
        out_shape=[
            jax.ShapeDtypeStruct((BC, 1, 128), jnp.float32),
            jax.ShapeDtypeStruct((BC, 1, 128), jnp.int32),
        ],
        compiler_params=pltpu.CompilerParams(
            dimension_semantics=("arbitrary",)),
    )(xr)
    maxs = maxs.reshape(BC, 128)
    idxs = idxs.reshape(BC, 128)

    stats, rv, cv = pl.pallas_call(
        _stats_kernel,
        out_shape=[
            jax.ShapeDtypeStruct((BC, 128), jnp.float32),
            jax.ShapeDtypeStruct((BC, H), jnp.float32),
            jax.ShapeDtypeStruct((BC, W), jnp.float32),
        ],
    )(maxs, idxs)

    att = pl.pallas_call(
        _outer_kernel,
        grid=(BC,),
        in_specs=[
            pl.BlockSpec((1, 1, H), lambda i: (i, 0, 0)),
            pl.BlockSpec((1, 1, W), lambda i: (i, 0, 0)),
        ],
        out_specs=pl.BlockSpec((1, H, W), lambda i: (i, 0, 0)),
        out_shape=jax.ShapeDtypeStruct((BC, H, W), jnp.float32),
        compiler_params=pltpu.CompilerParams(
            dimension_semantics=("arbitrary",)),
    )(rv.reshape(BC, 1, H), cv.reshape(BC, 1, W))

    att = att.reshape(B, C, H, W)
    m_joints = stats[:, 0:2].reshape(B, C, 2)
    v_joints = stats[:, 2:4].reshape(B, C, 2)
    vis = (stats[:, 4] > 0.5).reshape(B, C)
    return att, m_joints, v_joints, vis


def kernel(_x_):
    return _ips(_x_)


# speedup vs baseline: 1.3559x; 1.0000x over previous
import numpy as np
import jax
import jax.numpy as jnp
from jax.experimental import pallas as pl
from jax.experimental.pallas import tpu as pltpu

B, C, H, W = 4, 96, 224, 224
BC = B * C
HW = H * W
T = 5
KSIZE = 7
SIGMA = 1.5
PAD = KSIZE // 2


def _np_threefry2x32(k1, k2, x0, x1):
    x0 = np.uint32(x0)
    x1 = np.uint32(x1)
    ks0 = np.uint32(k1)
    ks1 = np.uint32(k2)
    ks2 = np.uint32(ks0 ^ ks1 ^ np.uint32(0x1BD11BDA))
    with np.errstate(over="ignore"):
        x0 = (x0 + ks0).astype(np.uint32)
        x1 = (x1 + ks1).astype(np.uint32)
        rots = [[13, 15, 26, 6], [17, 29, 16, 24]]
        kadd = [(ks1, ks2, 1), (ks2, ks0, 2), (ks0, ks1, 3),
                (ks1, ks2, 4), (ks2, ks0, 5)]
        for i in range(5):
            for r in rots[i % 2]:
                x0 = (x0 + x1).astype(np.uint32)
                x1 = ((x1 << np.uint32(r)) | (x1 >> np.uint32(32 - r))).astype(np.uint32)
                x1 = (x1 ^ x0).astype(np.uint32)
            a, b, c = kadd[i]
            x0 = (x0 + a).astype(np.uint32)
            x1 = (x1 + b + np.uint32(c)).astype(np.uint32)
    return x0, x1


_KEYS = [_np_threefry2x32(np.uint32(0), np.uint32(42), np.uint32(0), np.uint32(t))
         for t in range(T)]

_xs = np.linspace(-PAD, PAD, KSIZE, dtype=np.float32)
_pdf = np.exp(np.float32(-0.5) * (_xs / np.float32(SIGMA)) ** 2, dtype=np.float32)
_K1TAPS = (_pdf / _pdf.sum(dtype=np.float32)).astype(np.float32)


def _rotl(x, d):
    return (x << jnp.uint32(d)) | (x >> jnp.uint32(32 - d))


def _threefry_bits(k1, k2, lo):
    ks0 = jnp.uint32(k1)
    ks1 = jnp.uint32(k2)
    ks2 = jnp.uint32(int(np.uint32(k1) ^ np.uint32(k2) ^ np.uint32(0x1BD11BDA)))
    x0 = jnp.full_like(lo, ks0)
    x1 = lo + ks1
    rots = [[13, 15, 26, 6], [17, 29, 16, 24]]
    kadd = [(ks1, ks2, 1), (ks2, ks0, 2), (ks0, ks1, 3),
            (ks1, ks2, 4), (ks2, ks0, 5)]
    for i in range(5):
        for r in rots[i % 2]:
            x0 = x0 + x1
            x1 = _rotl(x1, r)
            x1 = x1 ^ x0
        a, b, c = kadd[i]
        x0 = x0 + a
        x1 = x1 + b + jnp.uint32(c)
    return x0 ^ x1


def _sample_kernel(x_ref, maxs_ref, idxs_ref):
    pid = pl.program_id(0)
    sig = jax.nn.sigmoid(x_ref[0])
    row = jax.lax.broadcasted_iota(jnp.int32, (H, W), 0)
    col = jax.lax.broadcasted_iota(jnp.int32, (H, W), 1)
    plane_idx = row * W + col
    lo = (pid * HW + plane_idx).astype(jnp.uint32)
    lane = jax.lax.broadcasted_iota(jnp.int32, (1, 128), 1)
    mvec = jnp.zeros((1, 128), jnp.float32)
    ivec = jnp.zeros((1, 128), jnp.int32)
    for t in range(T):
        k1, k2 = _KEYS[t]
        bits = _threefry_bits(int(k1), int(k2), lo)
        u = jax.lax.bitcast_convert_type(
            (bits >> jnp.uint32(9)) | jnp.uint32(0x3F800000), jnp.float32) - 1.0
        crnd = sig * u
        m_t = jnp.max(crnd)
        am_t = jnp.min(jnp.where(crnd == m_t, plane_idx, HW))
        mvec = jnp.where(lane == t, m_t, mvec)
        ivec = jnp.where(lane == t, am_t, ivec)
    maxs_ref[0] = mvec
    idxs_ref[0] = ivec


def _taps_at(a):
    out = jnp.zeros(a.shape, jnp.float32)
    for j in range(KSIZE):
        out = out + jnp.where(a == j, jnp.float32(_K1TAPS[j]), 0.0)
    return out


def _blur_vec(r, n):
    y = jax.lax.broadcasted_iota(jnp.int32, (BC, n), 1)
    v = _taps_at(r + PAD - y)
    top = (r >= 1) & (r <= PAD)
    v = v + jnp.where(top, _taps_at(PAD - r - y), 0.0)
    bot = (r >= n - 1 - PAD) & (r <= n - 2)
    v = v + jnp.where(bot, _taps_at(2 * n + 1 - r - y), 0.0)
    return v


def _stats_kernel(maxs_ref, idxs_ref, stats_ref, rv_ref, cv_ref):
    maxs = maxs_ref[...]
    idxs = idxs_ref[...]
    lane = jax.lax.broadcasted_iota(jnp.int32, (BC, 128), 1)
    tmask = lane < T
    vis_t = jnp.where(tmask & (maxs >= 0.1), 1.0, 0.0)
    i_t = (idxs // W).astype(jnp.float32) / H
    j_t = (idxs % W).astype(jnp.float32) / W
    yi = i_t * vis_t
    yj = j_t * vis_t
    m_i = jnp.sum(yi, axis=1, keepdims=True) / T
    m_j = jnp.sum(yj, axis=1, keepdims=True) / T
    di = jnp.where(tmask, yi - m_i, 0.0)
    dj = jnp.where(tmask, yj - m_j, 0.0)
    v_i = jnp.sum(di * di, axis=1, keepdims=True) / (T - 1)
    v_j = jnp.sum(dj * dj, axis=1, keepdims=True) / (T - 1)
    cnt = jnp.sum(vis_t, axis=1, keepdims=True)
    visf = jnp.where(cnt > 0.5 * T, 1.0, 0.0)
    m_i = m_i * visf
    m_j = m_j * visf
    v_i = v_i * visf
    v_j = v_j * visf
    rows = jnp.clip(jnp.round(m_i * H), 0, H - 1).astype(jnp.int32)
    cols = jnp.clip(jnp.round(m_j * W), 0, W - 1).astype(jnp.int32)
    rv = _blur_vec(rows, H)
    cv = _blur_vec(cols, W)
    chmax = visf * jnp.max(rv, axis=1, keepdims=True) * jnp.max(cv, axis=1, keepdims=True)
    gmax = jnp.max(chmax)
    inv = 1.0 / gmax
    rv_ref[...] = rv * visf * inv
    cv_ref[...] = cv
    stats = jnp.zeros((BC, 128), jnp.float32)
    for k, vec in enumerate((m_i, m_j, v_i, v_j, visf)):
        stats = jnp.where(lane == k, vec, stats)
    stats_ref[...] = stats


def _outer_kernel(rv_ref, cv_ref, att_ref):
    rcol = jnp.transpose(rv_ref[0])
    att_ref[0] = rcol * cv_ref[0]


@jax.jit
def _ips(x):
    xr = x.reshape(BC, H, W)
    maxs, idxs = pl.pallas_call(
        _sample_kernel,
        grid=(BC,),
        in_specs=[pl.BlockSpec((1, H, W), lambda i: (i, 0, 0))],
        out_specs=[
            pl.BlockSpec((1, 1, 128), lambda i: (i, 0, 0)),
            pl.BlockSpec((1, 1, 128), lambda i: (i, 0, 0)),
        ],
        out_shape=[
            jax.ShapeDtypeStruct((BC, 1, 128), jnp.float32),
            jax.ShapeDtypeStruct((BC, 1, 128), jnp.int32),
        ],
        compiler_params=pltpu.CompilerParams(
            dimension_semantics=("parallel",)),
    )(xr)
    maxs = maxs.reshape(BC, 128)
    idxs = idxs.reshape(BC, 128)

    stats, rv, cv = pl.pallas_call(
        _stats_kernel,
        out_shape=[
            jax.ShapeDtypeStruct((BC, 128), jnp.float32),
            jax.ShapeDtypeStruct((BC, H), jnp.float32),
            jax.ShapeDtypeStruct((BC, W), jnp.float32),
        ],
    )(maxs, idxs)

    att = pl.pallas_call(
        _outer_kernel,
        grid=(BC,),
        in_specs=[
            pl.BlockSpec((1, 1, H), lambda i: (i, 0, 0)),
            pl.BlockSpec((1, 1, W), lambda i: (i, 0, 0)),
        ],
        out_specs=pl.BlockSpec((1, H, W), lambda i: (i, 0, 0)),
        out_shape=jax.ShapeDtypeStruct((BC, H, W), jnp.float32),
        compiler_params=pltpu.CompilerParams(
            dimension_semantics=("parallel",)),
    )(rv.reshape(BC, 1, H), cv.reshape(BC, 1, W))

    att = att.reshape(B, C, H, W)
    m_joints = stats[:, 0:2].reshape(B, C, 2)
    v_joints = stats[:, 2:4].reshape(B, C, 2)
    vis = (stats[:, 4] > 0.5).reshape(B, C)
    return att, m_joints, v_joints, vis


def kernel(_x_):
    return _ips(_x_)


# speedup vs baseline: 1.5466x; 1.1406x over previous
import numpy as np
import jax
import jax.numpy as jnp
from jax.experimental import pallas as pl
from jax.experimental.pallas import tpu as pltpu

B, C, H, W = 4, 96, 224, 224
BC = B * C
HW = H * W
T = 5
KSIZE = 7
SIGMA = 1.5
PAD = KSIZE // 2


def _np_threefry2x32(k1, k2, x0, x1):
    x0 = np.uint32(x0)
    x1 = np.uint32(x1)
    ks0 = np.uint32(k1)
    ks1 = np.uint32(k2)
    ks2 = np.uint32(ks0 ^ ks1 ^ np.uint32(0x1BD11BDA))
    with np.errstate(over="ignore"):
        x0 = (x0 + ks0).astype(np.uint32)
        x1 = (x1 + ks1).astype(np.uint32)
        rots = [[13, 15, 26, 6], [17, 29, 16, 24]]
        kadd = [(ks1, ks2, 1), (ks2, ks0, 2), (ks0, ks1, 3),
                (ks1, ks2, 4), (ks2, ks0, 5)]
        for i in range(5):
            for r in rots[i % 2]:
                x0 = (x0 + x1).astype(np.uint32)
                x1 = ((x1 << np.uint32(r)) | (x1 >> np.uint32(32 - r))).astype(np.uint32)
                x1 = (x1 ^ x0).astype(np.uint32)
            a, b, c = kadd[i]
            x0 = (x0 + a).astype(np.uint32)
            x1 = (x1 + b + np.uint32(c)).astype(np.uint32)
    return x0, x1


_KEYS = [_np_threefry2x32(np.uint32(0), np.uint32(42), np.uint32(0), np.uint32(t))
         for t in range(T)]

_xs = np.linspace(-PAD, PAD, KSIZE, dtype=np.float32)
_pdf = np.exp(np.float32(-0.5) * (_xs / np.float32(SIGMA)) ** 2, dtype=np.float32)
_K1TAPS = (_pdf / _pdf.sum(dtype=np.float32)).astype(np.float32)


def _rotl(x, d):
    return (x << jnp.uint32(d)) | (x >> jnp.uint32(32 - d))


def _threefry_bits(k1, k2, lo):
    ks0 = jnp.uint32(k1)
    ks1 = jnp.uint32(k2)
    ks2 = jnp.uint32(int(np.uint32(k1) ^ np.uint32(k2) ^ np.uint32(0x1BD11BDA)))
    x0 = jnp.full_like(lo, ks0)
    x1 = lo + ks1
    rots = [[13, 15, 26, 6], [17, 29, 16, 24]]
    kadd = [(ks1, ks2, 1), (ks2, ks0, 2), (ks0, ks1, 3),
            (ks1, ks2, 4), (ks2, ks0, 5)]
    for i in range(5):
        for r in rots[i % 2]:
            x0 = x0 + x1
            x1 = _rotl(x1, r)
            x1 = x1 ^ x0
        a, b, c = kadd[i]
        x0 = x0 + a
        x1 = x1 + b + jnp.uint32(c)
    return x0 ^ x1


_TILE = 8
_NTILES = HW // (_TILE * 128)


def _sample_kernel(x_ref, maxs_ref, idxs_ref):
    pid = pl.program_id(0)
    sub = jax.lax.broadcasted_iota(jnp.int32, (_TILE, 128), 0)
    lane_col = jax.lax.broadcasted_iota(jnp.int32, (_TILE, 128), 1)
    tile_idx0 = sub * 128 + lane_col
    base = pid * HW

    def body(i, carry):
        vmaxs, vidxs = carry
        tile = x_ref[0, pl.ds(i * _TILE, _TILE), :]
        sig = jax.nn.sigmoid(tile)
        plane_idx = i * (_TILE * 128) + tile_idx0
        lo = (base + plane_idx).astype(jnp.uint32)
        new_vmaxs = []
        new_vidxs = []
        for t in range(T):
            k1, k2 = _KEYS[t]
            bits = _threefry_bits(int(k1), int(k2), lo)
            u = jax.lax.bitcast_convert_type(
                (bits >> jnp.uint32(9)) | jnp.uint32(0x3F800000), jnp.float32) - 1.0
            crnd = sig * u
            take = crnd > vmaxs[t]
            new_vidxs.append(jnp.where(take, plane_idx, vidxs[t]))
            new_vmaxs.append(jnp.maximum(vmaxs[t], crnd))
        return tuple(new_vmaxs), tuple(new_vidxs)

    init = (tuple(jnp.full((_TILE, 128), -jnp.inf, jnp.float32) for _ in range(T)),
            tuple(jnp.zeros((_TILE, 128), jnp.int32) for _ in range(T)))
    vmaxs, vidxs = jax.lax.fori_loop(0, _NTILES, body, init)

    lane = jax.lax.broadcasted_iota(jnp.int32, (1, 128), 1)
    mvec = jnp.zeros((1, 128), jnp.float32)
    ivec = jnp.zeros((1, 128), jnp.int32)
    for t in range(T):
        m_t = jnp.max(vmaxs[t])
        am_t = jnp.min(jnp.where(vmaxs[t] == m_t, vidxs[t], HW))
        mvec = jnp.where(lane == t, m_t, mvec)
        ivec = jnp.where(lane == t, am_t, ivec)
    maxs_ref[0] = mvec
    idxs_ref[0] = ivec


def _taps_at(a):
    out = jnp.zeros(a.shape, jnp.float32)
    for j in range(KSIZE):
        out = out + jnp.where(a == j, jnp.float32(_K1TAPS[j]), 0.0)
    return out


def _blur_vec(r, n):
    y = jax.lax.broadcasted_iota(jnp.int32, (BC, n), 1)
    v = _taps_at(r + PAD - y)
    top = (r >= 1) & (r <= PAD)
    v = v + jnp.where(top, _taps_at(PAD - r - y), 0.0)
    bot = (r >= n - 1 - PAD) & (r <= n - 2)
    v = v + jnp.where(bot, _taps_at(2 * n + 1 - r - y), 0.0)
    return v


def _stats_kernel(maxs_ref, idxs_ref, stats_ref, rv_ref, cv_ref):
    maxs = maxs_ref[...]
    idxs = idxs_ref[...]
    lane = jax.lax.broadcasted_iota(jnp.int32, (BC, 128), 1)
    tmask = lane < T
    vis_t = jnp.where(tmask & (maxs >= 0.1), 1.0, 0.0)
    i_t = (idxs // W).astype(jnp.float32) / H
    j_t = (idxs % W).astype(jnp.float32) / W
    yi = i_t * vis_t
    yj = j_t * vis_t
    m_i = jnp.sum(yi, axis=1, keepdims=True) / T
    m_j = jnp.sum(yj, axis=1, keepdims=True) / T
    di = jnp.where(tmask, yi - m_i, 0.0)
    dj = jnp.where(tmask, yj - m_j, 0.0)
    v_i = jnp.sum(di * di, axis=1, keepdims=True) / (T - 1)
    v_j = jnp.sum(dj * dj, axis=1, keepdims=True) / (T - 1)
    cnt = jnp.sum(vis_t, axis=1, keepdims=True)
    visf = jnp.where(cnt > 0.5 * T, 1.0, 0.0)
    m_i = m_i * visf
    m_j = m_j * visf
    v_i = v_i * visf
    v_j = v_j * visf
    rows = jnp.clip(jnp.round(m_i * H), 0, H - 1).astype(jnp.int32)
    cols = jnp.clip(jnp.round(m_j * W), 0, W - 1).astype(jnp.int32)
    rv = _blur_vec(rows, H)
    cv = _blur_vec(cols, W)
    chmax = visf * jnp.max(rv, axis=1, keepdims=True) * jnp.max(cv, axis=1, keepdims=True)
    gmax = jnp.max(chmax)
    inv = 1.0 / gmax
    rv_ref[...] = rv * visf * inv
    cv_ref[...] = cv
    stats = jnp.zeros((BC, 128), jnp.float32)
    for k, vec in enumerate((m_i, m_j, v_i, v_j, visf)):
        stats = jnp.where(lane == k, vec, stats)
    stats_ref[...] = stats


def _outer_kernel(rv_ref, cv_ref, att_ref):
    rcol = jnp.transpose(rv_ref[0])
    att_ref[0] = rcol * cv_ref[0]


@jax.jit
def _ips(x):
    xr = x.reshape(BC, HW // 128, 128)
    maxs, idxs = pl.pallas_call(
        _sample_kernel,
        grid=(BC,),
        in_specs=[pl.BlockSpec((1, HW // 128, 128), lambda i: (i, 0, 0))],
        out_specs=[
            pl.BlockSpec((1, 1, 128), lambda i: (i, 0, 0)),
            pl.BlockSpec((1, 1, 128), lambda i: (i, 0, 0)),
        ],
        out_shape=[
            jax.ShapeDtypeStruct((BC, 1, 128), jnp.float32),
            jax.ShapeDtypeStruct((BC, 1, 128), jnp.int32),
        ],
        compiler_params=pltpu.CompilerParams(
            dimension_semantics=("parallel",)),
    )(xr)
    maxs = maxs.reshape(BC, 128)
    idxs = idxs.reshape(BC, 128)

    stats, rv, cv = pl.pallas_call(
        _stats_kernel,
        out_shape=[
            jax.ShapeDtypeStruct((BC, 128), jnp.float32),
            jax.ShapeDtypeStruct((BC, H), jnp.float32),
            jax.ShapeDtypeStruct((BC, W), jnp.float32),
        ],
    )(maxs, idxs)

    att = pl.pallas_call(
        _outer_kernel,
        grid=(BC,),
        in_specs=[
            pl.BlockSpec((1, 1, H), lambda i: (i, 0, 0)),
            pl.BlockSpec((1, 1, W), lambda i: (i, 0, 0)),
        ],
        out_specs=pl.BlockSpec((1, H, W), lambda i: (i, 0, 0)),
        out_shape=jax.ShapeDtypeStruct((BC, H, W), jnp.float32),
        compiler_params=pltpu.CompilerParams(
            dimension_semantics=("parallel",)),
    )(rv.reshape(BC, 1, H), cv.reshape(BC, 1, W))

    att = att.reshape(B, C, H, W)
    m_joints = stats[:, 0:2].reshape(B, C, 2)
    v_joints = stats[:, 2:4].reshape(B, C, 2)
    vis = (stats[:, 4] > 0.5).reshape(B, C)
    return att, m_joints, v_joints, vis


def kernel(_x_):
    return _ips(_x_)


# speedup vs baseline: 1.8180x; 1.1755x over previous
import numpy as np
import jax
import jax.numpy as jnp
from jax.experimental import pallas as pl
from jax.experimental.pallas import tpu as pltpu

B, C, H, W = 4, 96, 224, 224
BC = B * C
HW = H * W
T = 5
KSIZE = 7
SIGMA = 1.5
PAD = KSIZE // 2


def _np_threefry2x32(k1, k2, x0, x1):
    x0 = np.uint32(x0)
    x1 = np.uint32(x1)
    ks0 = np.uint32(k1)
    ks1 = np.uint32(k2)
    ks2 = np.uint32(ks0 ^ ks1 ^ np.uint32(0x1BD11BDA))
    with np.errstate(over="ignore"):
        x0 = (x0 + ks0).astype(np.uint32)
        x1 = (x1 + ks1).astype(np.uint32)
        rots = [[13, 15, 26, 6], [17, 29, 16, 24]]
        kadd = [(ks1, ks2, 1), (ks2, ks0, 2), (ks0, ks1, 3),
                (ks1, ks2, 4), (ks2, ks0, 5)]
        for i in range(5):
            for r in rots[i % 2]:
                x0 = (x0 + x1).astype(np.uint32)
                x1 = ((x1 << np.uint32(r)) | (x1 >> np.uint32(32 - r))).astype(np.uint32)
                x1 = (x1 ^ x0).astype(np.uint32)
            a, b, c = kadd[i]
            x0 = (x0 + a).astype(np.uint32)
            x1 = (x1 + b + np.uint32(c)).astype(np.uint32)
    return x0, x1


_KEYS = [_np_threefry2x32(np.uint32(0), np.uint32(42), np.uint32(0), np.uint32(t))
         for t in range(T)]

_xs = np.linspace(-PAD, PAD, KSIZE, dtype=np.float32)
_pdf = np.exp(np.float32(-0.5) * (_xs / np.float32(SIGMA)) ** 2, dtype=np.float32)
_K1TAPS = (_pdf / _pdf.sum(dtype=np.float32)).astype(np.float32)


def _rotl(x, d):
    return (x << jnp.uint32(d)) | (x >> jnp.uint32(32 - d))


def _threefry_bits(k1, k2, lo):
    ks0 = jnp.uint32(k1)
    ks1 = jnp.uint32(k2)
    ks2 = jnp.uint32(int(np.uint32(k1) ^ np.uint32(k2) ^ np.uint32(0x1BD11BDA)))
    x0 = jnp.full_like(lo, ks0)
    x1 = lo + ks1
    rots = [[13, 15, 26, 6], [17, 29, 16, 24]]
    kadd = [(ks1, ks2, 1), (ks2, ks0, 2), (ks0, ks1, 3),
            (ks1, ks2, 4), (ks2, ks0, 5)]
    for i in range(5):
        for r in rots[i % 2]:
            x0 = x0 + x1
            x1 = _rotl(x1, r)
            x1 = x1 ^ x0
        a, b, c = kadd[i]
        x0 = x0 + a
        x1 = x1 + b + jnp.uint32(c)
    return x0 ^ x1


_TILE = 8
_NVREG = HW // (_TILE * 128)
_PAIRS = _NVREG // 2


def _sample_kernel(x_ref, maxs_ref, idxs_ref):
    pid = pl.program_id(0)
    sub = jax.lax.broadcasted_iota(jnp.int32, (_TILE, 128), 0)
    lane_col = jax.lax.broadcasted_iota(jnp.int32, (_TILE, 128), 1)
    tile_idx0 = sub * 128 + lane_col
    base = pid * HW

    def visit(tile_no, vmaxs, vidxs):
        tile = x_ref[0, pl.ds(tile_no * _TILE, _TILE), :]
        sig = jax.nn.sigmoid(tile)
        plane_idx = tile_no * (_TILE * 128) + tile_idx0
        lo = (base + plane_idx).astype(jnp.uint32)
        new_vmaxs = []
        new_vidxs = []
        for t in range(T):
            k1, k2 = _KEYS[t]
            bits = _threefry_bits(int(k1), int(k2), lo)
            u = jax.lax.bitcast_convert_type(
                (bits >> jnp.uint32(9)) | jnp.uint32(0x3F800000), jnp.float32) - 1.0
            crnd = sig * u
            take = crnd > vmaxs[t]
            new_vidxs.append(jnp.where(take, plane_idx, vidxs[t]))
            new_vmaxs.append(jnp.maximum(vmaxs[t], crnd))
        return tuple(new_vmaxs), tuple(new_vidxs)

    def body(i, carry):
        vmaxs0, vidxs0, vmaxs1, vidxs1 = carry
        vmaxs0, vidxs0 = visit(2 * i, vmaxs0, vidxs0)
        vmaxs1, vidxs1 = visit(2 * i + 1, vmaxs1, vidxs1)
        return vmaxs0, vidxs0, vmaxs1, vidxs1

    zmax = tuple(jnp.full((_TILE, 128), -jnp.inf, jnp.float32) for _ in range(T))
    zidx = tuple(jnp.zeros((_TILE, 128), jnp.int32) for _ in range(T))
    vmaxs0, vidxs0, vmaxs1, vidxs1 = jax.lax.fori_loop(
        0, _PAIRS, body, (zmax, zidx, zmax, zidx))
    vmaxs0, vidxs0 = visit(_NVREG - 1, vmaxs0, vidxs0)

    lane = jax.lax.broadcasted_iota(jnp.int32, (1, 128), 1)
    mvec = jnp.zeros((1, 128), jnp.float32)
    ivec = jnp.zeros((1, 128), jnp.int32)
    for t in range(T):
        m_t = jnp.maximum(jnp.max(vmaxs0[t]), jnp.max(vmaxs1[t]))
        am_t = jnp.minimum(
            jnp.min(jnp.where(vmaxs0[t] == m_t, vidxs0[t], HW)),
            jnp.min(jnp.where(vmaxs1[t] == m_t, vidxs1[t], HW)))
        mvec = jnp.where(lane == t, m_t, mvec)
        ivec = jnp.where(lane == t, am_t, ivec)
    maxs_ref[0] = mvec
    idxs_ref[0] = ivec


def _taps_at(a):
    out = jnp.zeros(a.shape, jnp.float32)
    for j in range(KSIZE):
        out = out + jnp.where(a == j, jnp.float32(_K1TAPS[j]), 0.0)
    return out


def _blur_vec(r, n):
    y = jax.lax.broadcasted_iota(jnp.int32, (BC, n), 1)
    v = _taps_at(r + PAD - y)
    top = (r >= 1) & (r <= PAD)
    v = v + jnp.where(top, _taps_at(PAD - r - y), 0.0)
    bot = (r >= n - 1 - PAD) & (r <= n - 2)
    v = v + jnp.where(bot, _taps_at(2 * n + 1 - r - y), 0.0)
    return v


def _stats_kernel(maxs_ref, idxs_ref, stats_ref, rv_ref, cv_ref):
    maxs = maxs_ref[...]
    idxs = idxs_ref[...]
    lane = jax.lax.broadcasted_iota(jnp.int32, (BC, 128), 1)
    tmask = lane < T
    vis_t = jnp.where(tmask & (maxs >= 0.1), 1.0, 0.0)
    i_t = (idxs // W).astype(jnp.float32) / H
    j_t = (idxs % W).astype(jnp.float32) / W
    yi = i_t * vis_t
    yj = j_t * vis_t
    m_i = jnp.sum(yi, axis=1, keepdims=True) / T
    m_j = jnp.sum(yj, axis=1, keepdims=True) / T
    di = jnp.where(tmask, yi - m_i, 0.0)
    dj = jnp.where(tmask, yj - m_j, 0.0)
    v_i = jnp.sum(di * di, axis=1, keepdims=True) / (T - 1)
    v_j = jnp.sum(dj * dj, axis=1, keepdims=True) / (T - 1)
    cnt = jnp.sum(vis_t, axis=1, keepdims=True)
    visf = jnp.where(cnt > 0.5 * T, 1.0, 0.0)
    m_i = m_i * visf
    m_j = m_j * visf
    v_i = v_i * visf
    v_j = v_j * visf
    rows = jnp.clip(jnp.round(m_i * H), 0, H - 1).astype(jnp.int32)
    cols = jnp.clip(jnp.round(m_j * W), 0, W - 1).astype(jnp.int32)
    rv = _blur_vec(rows, H)
    cv = _blur_vec(cols, W)
    chmax = visf * jnp.max(rv, axis=1, keepdims=True) * jnp.max(cv, axis=1, keepdims=True)
    gmax = jnp.max(chmax)
    inv = 1.0 / gmax
    rv_ref[...] = rv * visf * inv
    cv_ref[...] = cv
    stats = jnp.zeros((BC, 128), jnp.float32)
    for k, vec in enumerate((m_i, m_j, v_i, v_j, visf)):
        stats = jnp.where(lane == k, vec, stats)
    stats_ref[...] = stats


def _outer_kernel(rv_ref, cv_ref, att_ref):
    rcol = jnp.transpose(rv_ref[0])
    att_ref[0] = rcol * cv_ref[0]


@jax.jit
def _ips(x):
    xr = x.reshape(BC, HW // 128, 128)
    maxs, idxs = pl.pallas_call(
        _sample_kernel,
        grid=(BC,),
        in_specs=[pl.BlockSpec((1, HW // 128, 128), lambda i: (i, 0, 0))],
        out_specs=[
            pl.BlockSpec((1, 1, 128), lambda i: (i, 0, 0)),
            pl.BlockSpec((1, 1, 128), lambda i: (i, 0, 0)),
        ],
        out_shape=[
            jax.ShapeDtypeStruct((BC, 1, 128), jnp.float32),
            jax.ShapeDtypeStruct((BC, 1, 128), jnp.int32),
        ],
        compiler_params=pltpu.CompilerParams(
            dimension_semantics=("parallel",)),
    )(xr)
    maxs = maxs.reshape(BC, 128)
    idxs = idxs.reshape(BC, 128)

    stats, rv, cv = pl.pallas_call(
        _stats_kernel,
        out_shape=[
            jax.ShapeDtypeStruct((BC, 128), jnp.float32),
            jax.ShapeDtypeStruct((BC, H), jnp.float32),
            jax.ShapeDtypeStruct((BC, W), jnp.float32),
        ],
    )(maxs, idxs)

    att = pl.pallas_call(
        _outer_kernel,
        grid=(BC,),
        in_specs=[
            pl.BlockSpec((1, 1, H), lambda i: (i, 0, 0)),
            pl.BlockSpec((1, 1, W), lambda i: (i, 0, 0)),
        ],
        out_specs=pl.BlockSpec((1, H, W), lambda i: (i, 0, 0)),
        out_shape=jax.ShapeDtypeStruct((BC, H, W), jnp.float32),
        compiler_params=pltpu.CompilerParams(
            dimension_semantics=("parallel",)),
    )(rv.reshape(BC, 1, H), cv.reshape(BC, 1, W))

    att = att.reshape(B, C, H, W)
    m_joints = stats[:, 0:2].reshape(B, C, 2)
    v_joints = stats[:, 2:4].reshape(B, C, 2)
    vis = (stats[:, 4] > 0.5).reshape(B, C)
    return att, m_joints, v_joints, vis


def kernel(_x_):
    return _ips(_x_)


# speedup vs baseline: 1.9378x; 1.0659x over previous
import numpy as np
import jax
import jax.numpy as jnp
from jax.experimental import pallas as pl
from jax.experimental.pallas import tpu as pltpu

B, C, H, W = 4, 96, 224, 224
BC = B * C
HW = H * W
T = 5
KSIZE = 7
SIGMA = 1.5
PAD = KSIZE // 2


def _np_threefry2x32(k1, k2, x0, x1):
    x0 = np.uint32(x0)
    x1 = np.uint32(x1)
    ks0 = np.uint32(k1)
    ks1 = np.uint32(k2)
    ks2 = np.uint32(ks0 ^ ks1 ^ np.uint32(0x1BD11BDA))
    with np.errstate(over="ignore"):
        x0 = (x0 + ks0).astype(np.uint32)
        x1 = (x1 + ks1).astype(np.uint32)
        rots = [[13, 15, 26, 6], [17, 29, 16, 24]]
        kadd = [(ks1, ks2, 1), (ks2, ks0, 2), (ks0, ks1, 3),
                (ks1, ks2, 4), (ks2, ks0, 5)]
        for i in range(5):
            for r in rots[i % 2]:
                x0 = (x0 + x1).astype(np.uint32)
                x1 = ((x1 << np.uint32(r)) | (x1 >> np.uint32(32 - r))).astype(np.uint32)
                x1 = (x1 ^ x0).astype(np.uint32)
            a, b, c = kadd[i]
            x0 = (x0 + a).astype(np.uint32)
            x1 = (x1 + b + np.uint32(c)).astype(np.uint32)
    return x0, x1


_KEYS = [_np_threefry2x32(np.uint32(0), np.uint32(42), np.uint32(0), np.uint32(t))
         for t in range(T)]

_xs = np.linspace(-PAD, PAD, KSIZE, dtype=np.float32)
_pdf = np.exp(np.float32(-0.5) * (_xs / np.float32(SIGMA)) ** 2, dtype=np.float32)
_K1TAPS = (_pdf / _pdf.sum(dtype=np.float32)).astype(np.float32)


def _rotl(x, d):
    return (x << jnp.uint32(d)) | (x >> jnp.uint32(32 - d))


def _threefry_bits(k1, k2, lo):
    ks0 = jnp.uint32(k1)
    ks1 = jnp.uint32(k2)
    ks2 = jnp.uint32(int(np.uint32(k1) ^ np.uint32(k2) ^ np.uint32(0x1BD11BDA)))
    x0 = jnp.full_like(lo, ks0)
    x1 = lo + ks1
    rots = [[13, 15, 26, 6], [17, 29, 16, 24]]
    kadd = [(ks1, ks2, 1), (ks2, ks0, 2), (ks0, ks1, 3),
            (ks1, ks2, 4), (ks2, ks0, 5)]
    for i in range(5):
        for r in rots[i % 2]:
            x0 = x0 + x1
            x1 = _rotl(x1, r)
            x1 = x1 ^ x0
        a, b, c = kadd[i]
        x0 = x0 + a
        x1 = x1 + b + jnp.uint32(c)
    return x0 ^ x1


_TILE = 8
_NVREG = HW // (_TILE * 128)
_PAIRS = _NVREG // 2
_CH = 4


def _sample_kernel(x_ref, maxs_ref, idxs_ref):
    pid = pl.program_id(0)
    sub = jax.lax.broadcasted_iota(jnp.int32, (_TILE, 128), 0)
    lane_col = jax.lax.broadcasted_iota(jnp.int32, (_TILE, 128), 1)
    tile_idx0 = sub * 128 + lane_col
    lane = jax.lax.broadcasted_iota(jnp.int32, (1, 128), 1)

    for ch in range(_CH):
        _sample_one(x_ref, maxs_ref, idxs_ref, ch,
                    (pid * _CH + ch) * HW, tile_idx0, lane)


def _sample_one(x_ref, maxs_ref, idxs_ref, ch, base, tile_idx0, lane):
    def visit(tile_no, vmaxs, vidxs):
        tile = x_ref[ch, pl.ds(tile_no * _TILE, _TILE), :]
        sig = jax.nn.sigmoid(tile)
        plane_idx = tile_no * (_TILE * 128) + tile_idx0
        lo = (base + plane_idx).astype(jnp.uint32)
        new_vmaxs = []
        new_vidxs = []
        for t in range(T):
            k1, k2 = _KEYS[t]
            bits = _threefry_bits(int(k1), int(k2), lo)
            u = jax.lax.bitcast_convert_type(
                (bits >> jnp.uint32(9)) | jnp.uint32(0x3F800000), jnp.float32) - 1.0
            crnd = sig * u
            take = crnd > vmaxs[t]
            new_vidxs.append(jnp.where(take, plane_idx, vidxs[t]))
            new_vmaxs.append(jnp.maximum(vmaxs[t], crnd))
        return tuple(new_vmaxs), tuple(new_vidxs)

    def body(i, carry):
        vmaxs0, vidxs0, vmaxs1, vidxs1 = carry
        vmaxs0, vidxs0 = visit(2 * i, vmaxs0, vidxs0)
        vmaxs1, vidxs1 = visit(2 * i + 1, vmaxs1, vidxs1)
        return vmaxs0, vidxs0, vmaxs1, vidxs1

    zmax = tuple(jnp.full((_TILE, 128), -jnp.inf, jnp.float32) for _ in range(T))
    zidx = tuple(jnp.zeros((_TILE, 128), jnp.int32) for _ in range(T))
    vmaxs0, vidxs0, vmaxs1, vidxs1 = jax.lax.fori_loop(
        0, _PAIRS, body, (zmax, zidx, zmax, zidx))
    vmaxs0, vidxs0 = visit(_NVREG - 1, vmaxs0, vidxs0)

    mvec = jnp.zeros((1, 128), jnp.float32)
    ivec = jnp.zeros((1, 128), jnp.int32)
    for t in range(T):
        m_t = jnp.maximum(jnp.max(vmaxs0[t]), jnp.max(vmaxs1[t]))
        am_t = jnp.minimum(
            jnp.min(jnp.where(vmaxs0[t] == m_t, vidxs0[t], HW)),
            jnp.min(jnp.where(vmaxs1[t] == m_t, vidxs1[t], HW)))
        mvec = jnp.where(lane == t, m_t, mvec)
        ivec = jnp.where(lane == t, am_t, ivec)
    maxs_ref[ch] = mvec
    idxs_ref[ch] = ivec


def _taps_at(a):
    out = jnp.zeros(a.shape, jnp.float32)
    for j in range(KSIZE):
        out = out + jnp.where(a == j, jnp.float32(_K1TAPS[j]), 0.0)
    return out


def _blur_vec(r, n):
    y = jax.lax.broadcasted_iota(jnp.int32, (BC, n), 1)
    v = _taps_at(r + PAD - y)
    top = (r >= 1) & (r <= PAD)
    v = v + jnp.where(top, _taps_at(PAD - r - y), 0.0)
    bot = (r >= n - 1 - PAD) & (r <= n - 2)
    v = v + jnp.where(bot, _taps_at(2 * n + 1 - r - y), 0.0)
    return v


def _stats_kernel(maxs_ref, idxs_ref, stats_ref, rv_ref, cv_ref):
    maxs = maxs_ref[...]
    idxs = idxs_ref[...]
    lane = jax.lax.broadcasted_iota(jnp.int32, (BC, 128), 1)
    tmask = lane < T
    vis_t = jnp.where(tmask & (maxs >= 0.1), 1.0, 0.0)
    i_t = (idxs // W).astype(jnp.float32) / H
    j_t = (idxs % W).astype(jnp.float32) / W
    yi = i_t * vis_t
    yj = j_t * vis_t
    m_i = jnp.sum(yi, axis=1, keepdims=True) / T
    m_j = jnp.sum(yj, axis=1, keepdims=True) / T
    di = jnp.where(tmask, yi - m_i, 0.0)
    dj = jnp.where(tmask, yj - m_j, 0.0)
    v_i = jnp.sum(di * di, axis=1, keepdims=True) / (T - 1)
    v_j = jnp.sum(dj * dj, axis=1, keepdims=True) / (T - 1)
    cnt = jnp.sum(vis_t, axis=1, keepdims=True)
    visf = jnp.where(cnt > 0.5 * T, 1.0, 0.0)
    m_i = m_i * visf
    m_j = m_j * visf
    v_i = v_i * visf
    v_j = v_j * visf
    rows = jnp.clip(jnp.round(m_i * H), 0, H - 1).astype(jnp.int32)
    cols = jnp.clip(jnp.round(m_j * W), 0, W - 1).astype(jnp.int32)
    rv = _blur_vec(rows, H)
    cv = _blur_vec(cols, W)
    chmax = visf * jnp.max(rv, axis=1, keepdims=True) * jnp.max(cv, axis=1, keepdims=True)
    gmax = jnp.max(chmax)
    inv = 1.0 / gmax
    rv_ref[...] = rv * visf * inv
    cv_ref[...] = cv
    stats = jnp.zeros((BC, 128), jnp.float32)
    for k, vec in enumerate((m_i, m_j, v_i, v_j, visf)):
        stats = jnp.where(lane == k, vec, stats)
    stats_ref[...] = stats


def _outer_kernel(rv_ref, cv_ref, att_ref):
    for ch in range(_CH):
        rcol = jnp.transpose(rv_ref[ch])
        att_ref[ch] = rcol * cv_ref[ch]


@jax.jit
def _ips(x):
    xr = x.reshape(BC, HW // 128, 128)
    maxs, idxs = pl.pallas_call(
        _sample_kernel,
        grid=(BC // _CH,),
        in_specs=[pl.BlockSpec((_CH, HW // 128, 128), lambda i: (i, 0, 0))],
        out_specs=[
            pl.BlockSpec((_CH, 1, 128), lambda i: (i, 0, 0)),
            pl.BlockSpec((_CH, 1, 128), lambda i: (i, 0, 0)),
        ],
        out_shape=[
            jax.ShapeDtypeStruct((BC, 1, 128), jnp.float32),
            jax.ShapeDtypeStruct((BC, 1, 128), jnp.int32),
        ],
        compiler_params=pltpu.CompilerParams(
            dimension_semantics=("parallel",)),
    )(xr)
    maxs = maxs.reshape(BC, 128)
    idxs = idxs.reshape(BC, 128)

    stats, rv, cv = pl.pallas_call(
        _stats_kernel,
        out_shape=[
            jax.ShapeDtypeStruct((BC, 128), jnp.float32),
            jax.ShapeDtypeStruct((BC, H), jnp.float32),
            jax.ShapeDtypeStruct((BC, W), jnp.float32),
        ],
    )(maxs, idxs)

    att = pl.pallas_call(
        _outer_kernel,
        grid=(BC // _CH,),
        in_specs=[
            pl.BlockSpec((_CH, 1, H), lambda i: (i, 0, 0)),
            pl.BlockSpec((_CH, 1, W), lambda i: (i, 0, 0)),
        ],
        out_specs=pl.BlockSpec((_CH, H, W), lambda i: (i, 0, 0)),
        out_shape=jax.ShapeDtypeStruct((BC, H, W), jnp.float32),
        compiler_params=pltpu.CompilerParams(
            dimension_semantics=("parallel",)),
    )(rv.reshape(BC, 1, H), cv.reshape(BC, 1, W))

    att = att.reshape(B, C, H, W)
    m_joints = stats[:, 0:2].reshape(B, C, 2)
    v_joints = stats[:, 2:4].reshape(B, C, 2)
    vis = (stats[:, 4] > 0.5).reshape(B, C)
    return att, m_joints, v_joints, vis


def kernel(_x_):
    return _ips(_x_)


# speedup vs baseline: 1.9951x; 1.0296x over previous
import numpy as np
import jax
import jax.numpy as jnp
from jax.experimental import pallas as pl
from jax.experimental.pallas import tpu as pltpu

B, C, H, W = 4, 96, 224, 224
BC = B * C
HW = H * W
T = 5
KSIZE = 7
SIGMA = 1.5
PAD = KSIZE // 2


def _np_threefry2x32(k1, k2, x0, x1):
    x0 = np.uint32(x0)
    x1 = np.uint32(x1)
    ks0 = np.uint32(k1)
    ks1 = np.uint32(k2)
    ks2 = np.uint32(ks0 ^ ks1 ^ np.uint32(0x1BD11BDA))
    with np.errstate(over="ignore"):
        x0 = (x0 + ks0).astype(np.uint32)
        x1 = (x1 + ks1).astype(np.uint32)
        rots = [[13, 15, 26, 6], [17, 29, 16, 24]]
        kadd = [(ks1, ks2, 1), (ks2, ks0, 2), (ks0, ks1, 3),
                (ks1, ks2, 4), (ks2, ks0, 5)]
        for i in range(5):
            for r in rots[i % 2]:
                x0 = (x0 + x1).astype(np.uint32)
                x1 = ((x1 << np.uint32(r)) | (x1 >> np.uint32(32 - r))).astype(np.uint32)
                x1 = (x1 ^ x0).astype(np.uint32)
            a, b, c = kadd[i]
            x0 = (x0 + a).astype(np.uint32)
            x1 = (x1 + b + np.uint32(c)).astype(np.uint32)
    return x0, x1


_KEYS = [_np_threefry2x32(np.uint32(0), np.uint32(42), np.uint32(0), np.uint32(t))
         for t in range(T)]

_xs = np.linspace(-PAD, PAD, KSIZE, dtype=np.float32)
_pdf = np.exp(np.float32(-0.5) * (_xs / np.float32(SIGMA)) ** 2, dtype=np.float32)
_K1TAPS = (_pdf / _pdf.sum(dtype=np.float32)).astype(np.float32)


def _rotl(x, d):
    return (x << jnp.uint32(d)) | (x >> jnp.uint32(32 - d))


def _threefry_bits(k1, k2, lo):
    ks0 = jnp.uint32(k1)
    ks1 = jnp.uint32(k2)
    ks2 = jnp.uint32(int(np.uint32(k1) ^ np.uint32(k2) ^ np.uint32(0x1BD11BDA)))
    x0 = jnp.full_like(lo, ks0)
    x1 = lo + ks1
    rots = [[13, 15, 26, 6], [17, 29, 16, 24]]
    kadd = [(ks1, ks2, 1), (ks2, ks0, 2), (ks0, ks1, 3),
            (ks1, ks2, 4), (ks2, ks0, 5)]
    for i in range(5):
        for r in rots[i % 2]:
            x0 = x0 + x1
            x1 = _rotl(x1, r)
            x1 = x1 ^ x0
        a, b, c = kadd[i]
        x0 = x0 + a
        x1 = x1 + b + jnp.uint32(c)
    return x0 ^ x1


_TILE = 8
_NVREG = HW // (_TILE * 128)
_PAIRS = _NVREG // 2
_CH = 4


def _sample_kernel(x_ref, maxs_ref, idxs_ref):
    pid = pl.program_id(0)
    sub = jax.lax.broadcasted_iota(jnp.int32, (_TILE, 128), 0)
    lane_col = jax.lax.broadcasted_iota(jnp.int32, (_TILE, 128), 1)
    tile_idx0 = sub * 128 + lane_col
    lane = jax.lax.broadcasted_iota(jnp.int32, (1, 128), 1)

    for ch in range(_CH):
        _sample_one(x_ref, maxs_ref, idxs_ref, ch,
                    (pid * _CH + ch) * HW, tile_idx0, lane)


def _sample_one(x_ref, maxs_ref, idxs_ref, ch, base, tile_idx0, lane):
    def visit(tile_no, vmaxs, vidxs):
        tile = x_ref[ch, pl.ds(tile_no * _TILE, _TILE), :]
        sig = jax.nn.sigmoid(tile)
        plane_idx = tile_no * (_TILE * 128) + tile_idx0
        lo = (base + plane_idx).astype(jnp.uint32)
        new_vmaxs = []
        new_vidxs = []
        for t in range(T):
            k1, k2 = _KEYS[t]
            bits = _threefry_bits(int(k1), int(k2), lo)
            u = jax.lax.bitcast_convert_type(
                (bits >> jnp.uint32(9)) | jnp.uint32(0x3F800000), jnp.float32) - 1.0
            crnd = sig * u
            take = crnd > vmaxs[t]
            new_vidxs.append(jnp.where(take, plane_idx, vidxs[t]))
            new_vmaxs.append(jnp.maximum(vmaxs[t], crnd))
        return tuple(new_vmaxs), tuple(new_vidxs)

    def body(i, carry):
        vmaxs0, vidxs0, vmaxs1, vidxs1, vmaxs2, vidxs2 = carry
        vmaxs0, vidxs0 = visit(3 * i, vmaxs0, vidxs0)
        vmaxs1, vidxs1 = visit(3 * i + 1, vmaxs1, vidxs1)
        vmaxs2, vidxs2 = visit(3 * i + 2, vmaxs2, vidxs2)
        return vmaxs0, vidxs0, vmaxs1, vidxs1, vmaxs2, vidxs2

    zmax = tuple(jnp.full((_TILE, 128), -jnp.inf, jnp.float32) for _ in range(T))
    zidx = tuple(jnp.zeros((_TILE, 128), jnp.int32) for _ in range(T))
    vmaxs0, vidxs0, vmaxs1, vidxs1, vmaxs2, vidxs2 = jax.lax.fori_loop(
        0, _NVREG // 3, body, (zmax, zidx, zmax, zidx, zmax, zidx))
    vmaxs0, vidxs0 = visit(_NVREG - 1, vmaxs0, vidxs0)

    mvec = jnp.zeros((1, 128), jnp.float32)
    ivec = jnp.zeros((1, 128), jnp.int32)
    for t in range(T):
        m_t = jnp.maximum(jnp.maximum(jnp.max(vmaxs0[t]), jnp.max(vmaxs1[t])),
                          jnp.max(vmaxs2[t]))
        am_t = jnp.minimum(
            jnp.minimum(
                jnp.min(jnp.where(vmaxs0[t] == m_t, vidxs0[t], HW)),
                jnp.min(jnp.where(vmaxs1[t] == m_t, vidxs1[t], HW))),
            jnp.min(jnp.where(vmaxs2[t] == m_t, vidxs2[t], HW)))
        mvec = jnp.where(lane == t, m_t, mvec)
        ivec = jnp.where(lane == t, am_t, ivec)
    maxs_ref[ch] = mvec
    idxs_ref[ch] = ivec


def _taps_at(a):
    out = jnp.zeros(a.shape, jnp.float32)
    for j in range(KSIZE):
        out = out + jnp.where(a == j, jnp.float32(_K1TAPS[j]), 0.0)
    return out


def _blur_vec(r, n):
    y = jax.lax.broadcasted_iota(jnp.int32, (BC, n), 1)
    v = _taps_at(r + PAD - y)
    top = (r >= 1) & (r <= PAD)
    v = v + jnp.where(top, _taps_at(PAD - r - y), 0.0)
    bot = (r >= n - 1 - PAD) & (r <= n - 2)
    v = v + jnp.where(bot, _taps_at(2 * n + 1 - r - y), 0.0)
    return v


def _stats_kernel(maxs_ref, idxs_ref, stats_ref, rv_ref, cv_ref):
    maxs = maxs_ref[...]
    idxs = idxs_ref[...]
    lane = jax.lax.broadcasted_iota(jnp.int32, (BC, 128), 1)
    tmask = lane < T
    vis_t = jnp.where(tmask & (maxs >= 0.1), 1.0, 0.0)
    i_t = (idxs // W).astype(jnp.float32) / H
    j_t = (idxs % W).astype(jnp.float32) / W
    yi = i_t * vis_t
    yj = j_t * vis_t
    m_i = jnp.sum(yi, axis=1, keepdims=True) / T
    m_j = jnp.sum(yj, axis=1, keepdims=True) / T
    di = jnp.where(tmask, yi - m_i, 0.0)
    dj = jnp.where(tmask, yj - m_j, 0.0)
    v_i = jnp.sum(di * di, axis=1, keepdims=True) / (T - 1)
    v_j = jnp.sum(dj * dj, axis=1, keepdims=True) / (T - 1)
    cnt = jnp.sum(vis_t, axis=1, keepdims=True)
    visf = jnp.where(cnt > 0.5 * T, 1.0, 0.0)
    m_i = m_i * visf
    m_j = m_j * visf
    v_i = v_i * visf
    v_j = v_j * visf
    rows = jnp.clip(jnp.round(m_i * H), 0, H - 1).astype(jnp.int32)
    cols = jnp.clip(jnp.round(m_j * W), 0, W - 1).astype(jnp.int32)
    rv = _blur_vec(rows, H)
    cv = _blur_vec(cols, W)
    chmax = visf * jnp.max(rv, axis=1, keepdims=True) * jnp.max(cv, axis=1, keepdims=True)
    gmax = jnp.max(chmax)
    inv = 1.0 / gmax
    rv_ref[...] = rv * visf * inv
    cv_ref[...] = cv
    stats = jnp.zeros((BC, 128), jnp.float32)
    for k, vec in enumerate((m_i, m_j, v_i, v_j, visf)):
        stats = jnp.where(lane == k, vec, stats)
    stats_ref[...] = stats


def _outer_kernel(rv_ref, cv_ref, att_ref):
    for ch in range(_CH):
        rcol = jnp.transpose(rv_ref[ch])
        att_ref[ch] = rcol * cv_ref[ch]


@jax.jit
def _ips(x):
    xr = x.reshape(BC, HW // 128, 128)
    maxs, idxs = pl.pallas_call(
        _sample_kernel,
        grid=(BC // _CH,),
        in_specs=[pl.BlockSpec((_CH, HW // 128, 128), lambda i: (i, 0, 0))],
        out_specs=[
            pl.BlockSpec((_CH, 1, 128), lambda i: (i, 0, 0)),
            pl.BlockSpec((_CH, 1, 128), lambda i: (i, 0, 0)),
        ],
        out_shape=[
            jax.ShapeDtypeStruct((BC, 1, 128), jnp.float32),
            jax.ShapeDtypeStruct((BC, 1, 128), jnp.int32),
        ],
        compiler_params=pltpu.CompilerParams(
            dimension_semantics=("parallel",)),
    )(xr)
    maxs = maxs.reshape(BC, 128)
    idxs = idxs.reshape(BC, 128)

    stats, rv, cv = pl.pallas_call(
        _stats_kernel,
        out_shape=[
            jax.ShapeDtypeStruct((BC, 128), jnp.float32),
            jax.ShapeDtypeStruct((BC, H), jnp.float32),
            jax.ShapeDtypeStruct((BC, W), jnp.float32),
        ],
    )(maxs, idxs)

    att = pl.pallas_call(
        _outer_kernel,
        grid=(BC // _CH,),
        in_specs=[
            pl.BlockSpec((_CH, 1, H), lambda i: (i, 0, 0)),
            pl.BlockSpec((_CH, 1, W), lambda i: (i, 0, 0)),
        ],
        out_specs=pl.BlockSpec((_CH, H, W), lambda i: (i, 0, 0)),
        out_shape=jax.ShapeDtypeStruct((BC, H, W), jnp.float32),
        compiler_params=pltpu.CompilerParams(
            dimension_semantics=("parallel",)),
    )(rv.reshape(BC, 1, H), cv.reshape(BC, 1, W))

    att = att.reshape(B, C, H, W)
    m_joints = stats[:, 0:2].reshape(B, C, 2)
    v_joints = stats[:, 2:4].reshape(B, C, 2)
    vis = (stats[:, 4] > 0.5).reshape(B, C)
    return att, m_joints, v_joints, vis


def kernel(_x_):
    return _ips(_x_)
